# baseline (speedup 1.0000x reference)
import functools
import math

import jax
import jax.numpy as jnp
from jax import lax
from jax.experimental import pallas as pl
from jax.experimental.pallas import tpu as pltpu
from jax.experimental.pallas import tpu_sc as plsc

V, D, B, C, NS = 100000, 64, 16384, 10, 20
NW = 32
BPW = B // NW
NG = BPW // 16
PRIOR_CONST = 2.0 * V * D * (-0.5 * math.log(2.0 * math.pi))


ROWS_PW = V // NW
PCHUNK = 125
NCHUNK = ROWS_PW // PCHUNK


def _sc_logits_body(w_word, w_ctx, tgt_hbm, ctx2_hbm, neg2_hbm,
                    pos_hbm, neg_hbm, prior_hbm,
                    tgt_idx,
                    t_3d, c_3d, n_3d, pos_v, neg_v, pr_3d, pr_v,
                    ci_3d, ni_3d, sc_3d, sn_3d,
                    sem0, sem1, semi0, semi1):
    wid = lax.axis_index("s") * 2 + lax.axis_index("c")
    base = wid * BPW
    pltpu.sync_copy(tgt_hbm.at[pl.ds(base, BPW)], tgt_idx)

    lane = lax.broadcasted_iota(jnp.int32, (16,), 0)
    c_row0 = lane * C
    n_row0 = lane * NS
    zero = jnp.zeros((16,), jnp.float32)

    bufs = ((t_3d.at[0], c_3d.at[0], n_3d.at[0], sem0),
            (t_3d.at[1], c_3d.at[1], n_3d.at[1], sem1))

    def descs(g, slot):
        t_b, c_b, n_b, sem = bufs[slot]
        o16 = pl.multiple_of(g * 16, 16)
        sc_ = sc_3d.at[slot]
        sn_ = sn_3d.at[slot]
        return (
            pltpu.make_async_copy(w_word.at[tgt_idx.at[pl.ds(o16, 16)]], t_b, sem),
            pltpu.make_async_copy(w_ctx.at[sc_.at[pl.ds(0, 80)]],
                                  c_b.at[pl.ds(0, 80)], sem),
            pltpu.make_async_copy(w_ctx.at[sc_.at[pl.ds(80, 80)]],
                                  c_b.at[pl.ds(80, 80)], sem),
            pltpu.make_async_copy(w_word.at[sn_.at[pl.ds(0, 128)]],
                                  n_b.at[pl.ds(0, 128)], sem),
            pltpu.make_async_copy(w_word.at[sn_.at[pl.ds(128, 128)]],
                                  n_b.at[pl.ds(128, 128)], sem),
            pltpu.make_async_copy(w_word.at[sn_.at[pl.ds(256, 64)]],
                                  n_b.at[pl.ds(256, 64)], sem),
        )

    def issue(g, slot):
        for cp in descs(g, slot):
            cp.start()

    def wait(g, slot):
        for cp in descs(g, slot):
            cp.wait()

    def idx_descs(g, slot):
        b0 = base + g * 16
        sem = semi0 if slot == 0 else semi1
        return (
            pltpu.make_async_copy(ctx2_hbm.at[pl.ds(b0, 16), :],
                                  ci_3d.at[slot], sem),
            pltpu.make_async_copy(neg2_hbm.at[pl.ds(b0, 16), :],
                                  ni_3d.at[slot], sem),
        )

    def issue_idx(g, slot):
        for cp in idx_descs(g, slot):
            cp.start()

    def wait_idx(g, slot):
        for cp in idx_descs(g, slot):
            cp.wait()

    c_rc = []
    for q in range(10):
        p = lane + q * 16
        c_rc.append((p // C, p % C))
    n_rc = []
    for q in range(20):
        p = lane + q * 16
        n_rc.append((p // NS, p % NS))

    def stage(g, slot):
        ci = ci_3d.at[slot]
        ni = ni_3d.at[slot]
        sc_ = sc_3d.at[slot]
        sn_ = sn_3d.at[slot]
        for q in range(10):
            r, cc = c_rc[q]
            sc_[pl.ds(q * 16, 16)] = plsc.load_gather(ci, [r, cc])
        for q in range(20):
            r, cc = n_rc[q]
            sn_[pl.ds(q * 16, 16)] = plsc.load_gather(ni, [r, cc])

    NV = D // 16

    def compute(g, slot):
        t_b, c_b, n_b, _ = bufs[slot]
        o16 = pl.multiple_of(g * 16, 16)

        def per_b(bb, carry):
            p_pack, n_pack = carry
            m = lane == jnp.full((16,), bb, jnp.int32)
            crow = bb * C
            nrow = bb * NS
            c_sum = []
            for k4 in range(NV):
                acc = c_b[crow, pl.ds(k4 * 16, 16)]
                for j in range(1, C):
                    acc = acc + c_b[crow + j, pl.ds(k4 * 16, 16)]
                c_sum.append(acc)
            pv = t_b[bb, pl.ds(0, 16)] * c_sum[0]
            for k4 in range(1, NV):
                pv = pv + t_b[bb, pl.ds(k4 * 16, 16)] * c_sum[k4]
            p_pack = jnp.where(m, jnp.sum(pv), p_pack)
            new_n = []
            for k in range(NS):
                nv = n_b[nrow + k, pl.ds(0, 16)] * c_sum[0]
                for k4 in range(1, NV):
                    nv = nv + n_b[nrow + k, pl.ds(k4 * 16, 16)] * c_sum[k4]
                new_n.append(jnp.where(m, jnp.sum(nv), n_pack[k]))
            return p_pack, tuple(new_n)

        p, n = lax.fori_loop(0, 16, per_b, (zero, (zero,) * NS))
        pos_v[pl.ds(o16, 16)] = p
        for k in range(NS):
            neg_v[pl.ds(k * BPW + o16, 16)] = n[k]

    issue_idx(0, 0)
    issue_idx(1, 1)
    wait_idx(0, 0)
    stage(0, 0)
    issue(0, 0)
    issue_idx(2, 0)
    wait_idx(1, 1)
    stage(1, 1)
    issue(1, 1)
    issue_idx(3, 1)

    def group_pair(it, _):
        g0 = it * 2
        wait(g0, 0)
        compute(g0, 0)

        @pl.when(g0 + 2 < NG)
        def _():
            wait_idx(g0 + 2, 0)
            stage(g0 + 2, 0)
            issue(g0 + 2, 0)

            @pl.when(g0 + 4 < NG)
            def _():
                issue_idx(g0 + 4, 0)

        wait(g0 + 1, 1)
        compute(g0 + 1, 1)

        @pl.when(g0 + 3 < NG)
        def _():
            wait_idx(g0 + 3, 1)
            stage(g0 + 3, 1)
            issue(g0 + 3, 1)

            @pl.when(g0 + 5 < NG)
            def _():
                issue_idx(g0 + 5, 1)

        return ()

    lax.fori_loop(0, NG // 2, group_pair, ())
    pltpu.sync_copy(pos_v, pos_hbm.at[pl.ds(base, BPW)])
    pltpu.sync_copy(neg_v, neg_hbm.at[pl.ds(wid * (BPW * NS), BPW * NS)])

    r0 = wid * ROWS_PW

    def prior_issue(c, slot):
        tbl = w_word if c < NCHUNK else w_ctx
        roff = r0 + (c % NCHUNK) * PCHUNK
        sem = sem0 if slot == 0 else sem1
        return pltpu.make_async_copy(tbl.at[pl.ds(roff, PCHUNK), :],
                                     pr_3d.at[slot], sem)

    def prior_sum(slot, acc):
        pr_b = pr_3d.at[slot]

        def row16(i, a):
            v = pr_b[i, pl.ds(0, 16)]
            a0 = a[0] + v * v
            v = pr_b[i, pl.ds(16, 16)]
            a1 = a[1] + v * v
            v = pr_b[i, pl.ds(32, 16)]
            a2 = a[2] + v * v
            v = pr_b[i, pl.ds(48, 16)]
            a3 = a[3] + v * v
            return (a0, a1, a2, a3)

        return lax.fori_loop(0, PCHUNK, row16, acc)

    for c in range(2 * NCHUNK):
        if c == 0:
            prior_issue(0, 0).start()
        if c + 1 < 2 * NCHUNK:
            prior_issue(c + 1, (c + 1) % 2).start()
        prior_issue(c, c % 2).wait()
        if c == 0:
            acc = ((zero,) * 4)
        acc = prior_sum(c % 2, acc)
    pr_v[pl.ds(0, 16)] = acc[0] + acc[1] + acc[2] + acc[3]
    pltpu.sync_copy(pr_v, prior_hbm.at[pl.ds(wid * 16, 16)])


_sc_logits = pl.kernel(
    _sc_logits_body,
    out_type=[jax.ShapeDtypeStruct((B,), jnp.float32),
              jax.ShapeDtypeStruct((B * NS,), jnp.float32),
              jax.ShapeDtypeStruct((NW * 16,), jnp.float32)],
    mesh=plsc.VectorSubcoreMesh(core_axis_name="c", subcore_axis_name="s"),
    compiler_params=pltpu.CompilerParams(needs_layout_passes=False, use_tc_tiling_on_sc=False),
    scratch_types=[
        pltpu.VMEM((BPW,), jnp.int32),
        pltpu.VMEM((2, 16, D), jnp.float32),
        pltpu.VMEM((2, 160, D), jnp.float32),
        pltpu.VMEM((2, 320, D), jnp.float32),
        pltpu.VMEM((BPW,), jnp.float32),
        pltpu.VMEM((BPW * NS,), jnp.float32),
        pltpu.VMEM((2, PCHUNK, D), jnp.float32),
        pltpu.VMEM((16,), jnp.float32),
        pltpu.VMEM((2, 16, C), jnp.int32),
        pltpu.VMEM((2, 16, NS), jnp.int32),
        pltpu.VMEM((2, 160), jnp.int32),
        pltpu.VMEM((2, 320), jnp.int32),
        pltpu.SemaphoreType.DMA,
        pltpu.SemaphoreType.DMA,
        pltpu.SemaphoreType.DMA,
        pltpu.SemaphoreType.DMA,
    ],
)

def _logsig(x):
    return jnp.minimum(x, 0.0) - jnp.log1p(jnp.exp(-jnp.abs(x)))


def _loss_body(pos_ref, neg_ref, pr_ref, o_ref):
    o_ref[0, 0] = (jnp.sum(_logsig(pos_ref[...]))
                   + jnp.sum(_logsig(-neg_ref[...]))
                   + (-0.5) * jnp.sum(pr_ref[...])
                   + jnp.float32(PRIOR_CONST))


def kernel(target_ixs, context_ixs, negative_sample_ixs, W_word, W_ctx):
    pos, neg, pr = _sc_logits(
        W_word, W_ctx,
        target_ixs.astype(jnp.int32),
        context_ixs.astype(jnp.int32),
        negative_sample_ixs.astype(jnp.int32),
    )
    ll = pl.pallas_call(
        _loss_body,
        out_specs=pl.BlockSpec(memory_space=pltpu.SMEM),
        out_shape=jax.ShapeDtypeStruct((1, 1), jnp.float32),
    )(pos.reshape(128, 128), neg.reshape(2560, 128), pr.reshape(4, 128))
    return ll[0, 0]


# speedup vs baseline: 1.0573x; 1.0573x over previous
import functools
import math

import jax
import jax.numpy as jnp
from jax import lax
from jax.experimental import pallas as pl
from jax.experimental.pallas import tpu as pltpu
from jax.experimental.pallas import tpu_sc as plsc

V, D, B, C, NS = 100000, 64, 16384, 10, 20
NW = 32
BPW = B // NW
NG = BPW // 16
PRIOR_CONST = 2.0 * V * D * (-0.5 * math.log(2.0 * math.pi))


ROWS_PW = V // NW
PCHUNK = 125
NCHUNK = ROWS_PW // PCHUNK


def _sc_prior_body(w_word, w_ctx, prior_hbm, pr_3d, pr_v, sem0, sem1):
    wid = lax.axis_index("s") * 2 + lax.axis_index("c")
    zero = jnp.zeros((16,), jnp.float32)
    r0 = wid * ROWS_PW

    def prior_issue(c, slot):
        tbl = w_word if c < NCHUNK else w_ctx
        roff = r0 + (c % NCHUNK) * PCHUNK
        sem = sem0 if slot == 0 else sem1
        return pltpu.make_async_copy(tbl.at[pl.ds(roff, PCHUNK), :],
                                     pr_3d.at[slot], sem)

    def prior_sum(slot, acc):
        pr_b = pr_3d.at[slot]

        def row16(i, a):
            v = pr_b[i, pl.ds(0, 16)]
            a0 = a[0] + v * v
            v = pr_b[i, pl.ds(16, 16)]
            a1 = a[1] + v * v
            v = pr_b[i, pl.ds(32, 16)]
            a2 = a[2] + v * v
            v = pr_b[i, pl.ds(48, 16)]
            a3 = a[3] + v * v
            return (a0, a1, a2, a3)

        return lax.fori_loop(0, PCHUNK, row16, acc)

    acc = ((zero,) * 4)
    for c in range(2 * NCHUNK):
        if c == 0:
            prior_issue(0, 0).start()
        if c + 1 < 2 * NCHUNK:
            prior_issue(c + 1, (c + 1) % 2).start()
        prior_issue(c, c % 2).wait()
        acc = prior_sum(c % 2, acc)
    pr_v[pl.ds(0, 16)] = acc[0] + acc[1] + acc[2] + acc[3]
    pltpu.sync_copy(pr_v, prior_hbm.at[pl.ds(wid * 16, 16)])


_sc_prior = pl.kernel(
    _sc_prior_body,
    out_type=[jax.ShapeDtypeStruct((NW * 16,), jnp.float32)],
    mesh=plsc.VectorSubcoreMesh(core_axis_name="c", subcore_axis_name="s"),
    compiler_params=pltpu.CompilerParams(needs_layout_passes=False, use_tc_tiling_on_sc=False),
    scratch_types=[
        pltpu.VMEM((2, PCHUNK, D), jnp.float32),
        pltpu.VMEM((16,), jnp.float32),
        pltpu.SemaphoreType.DMA,
        pltpu.SemaphoreType.DMA,
    ],
)


def _sc_logits_body(w_word, w_ctx, tgt_hbm, ctxf_hbm, negf_hbm,
                    pos_hbm, neg_hbm,
                    tgt_idx, ctx_idx, neg_idx,
                    t_3d, c_3d, n_3d, pos_v, neg_v, sem0, sem1):
    wid = lax.axis_index("s") * 2 + lax.axis_index("c")
    base = wid * BPW
    pltpu.sync_copy(tgt_hbm.at[pl.ds(base, BPW)], tgt_idx)
    pltpu.sync_copy(ctxf_hbm.at[pl.ds(wid * (BPW * C), BPW * C)], ctx_idx)
    pltpu.sync_copy(negf_hbm.at[pl.ds(wid * (BPW * NS), BPW * NS)], neg_idx)

    lane = lax.broadcasted_iota(jnp.int32, (16,), 0)
    c_row0 = lane * C
    n_row0 = lane * NS
    zero = jnp.zeros((16,), jnp.float32)

    bufs = ((t_3d.at[0], c_3d.at[0], n_3d.at[0], sem0),
            (t_3d.at[1], c_3d.at[1], n_3d.at[1], sem1))

    def descs(g, slot):
        t_b, c_b, n_b, sem = bufs[slot]
        o16 = pl.multiple_of(g * 16, 16)
        o160 = pl.multiple_of(g * 160, 32)
        o320 = pl.multiple_of(g * 320, 64)
        return (
            pltpu.make_async_copy(w_word.at[tgt_idx.at[pl.ds(o16, 16)]], t_b, sem),
            pltpu.make_async_copy(w_ctx.at[ctx_idx.at[pl.ds(o160, 80)]],
                                  c_b.at[pl.ds(0, 80)], sem),
            pltpu.make_async_copy(w_ctx.at[ctx_idx.at[pl.ds(o160 + 80, 80)]],
                                  c_b.at[pl.ds(80, 80)], sem),
            pltpu.make_async_copy(w_word.at[neg_idx.at[pl.ds(o320, 128)]],
                                  n_b.at[pl.ds(0, 128)], sem),
            pltpu.make_async_copy(w_word.at[neg_idx.at[pl.ds(o320 + 128, 128)]],
                                  n_b.at[pl.ds(128, 128)], sem),
            pltpu.make_async_copy(w_word.at[neg_idx.at[pl.ds(o320 + 256, 64)]],
                                  n_b.at[pl.ds(256, 64)], sem),
        )

    def issue(g, slot):
        for cp in descs(g, slot):
            cp.start()

    def wait(g, slot):
        for cp in descs(g, slot):
            cp.wait()

    NV = D // 16

    def compute(g, slot):
        t_b, c_b, n_b, _ = bufs[slot]
        o16 = pl.multiple_of(g * 16, 16)

        def per_b(bb, carry):
            p_pack, n_pack = carry
            m = lane == jnp.full((16,), bb, jnp.int32)
            crow = bb * C
            nrow = bb * NS
            c_sum = []
            for k4 in range(NV):
                acc = c_b[crow, pl.ds(k4 * 16, 16)]
                for j in range(1, C):
                    acc = acc + c_b[crow + j, pl.ds(k4 * 16, 16)]
                c_sum.append(acc)
            pv = t_b[bb, pl.ds(0, 16)] * c_sum[0]
            for k4 in range(1, NV):
                pv = pv + t_b[bb, pl.ds(k4 * 16, 16)] * c_sum[k4]
            p_pack = jnp.where(m, jnp.sum(pv), p_pack)
            new_n = []
            for k in range(NS):
                nv = n_b[nrow + k, pl.ds(0, 16)] * c_sum[0]
                for k4 in range(1, NV):
                    nv = nv + n_b[nrow + k, pl.ds(k4 * 16, 16)] * c_sum[k4]
                new_n.append(jnp.where(m, jnp.sum(nv), n_pack[k]))
            return p_pack, tuple(new_n)

        p, n = lax.fori_loop(0, 16, per_b, (zero, (zero,) * NS))
        pos_v[pl.ds(o16, 16)] = p
        for k in range(NS):
            neg_v[pl.ds(k * BPW + o16, 16)] = n[k]

    issue(0, 0)
    issue(1, 1)

    def group_pair(it, _):
        g0 = it * 2
        wait(g0, 0)
        compute(g0, 0)

        @pl.when(g0 + 2 < NG)
        def _():
            issue(g0 + 2, 0)

        wait(g0 + 1, 1)
        compute(g0 + 1, 1)

        @pl.when(g0 + 3 < NG)
        def _():
            issue(g0 + 3, 1)

        return ()

    lax.fori_loop(0, NG // 2, group_pair, ())
    pltpu.sync_copy(pos_v, pos_hbm.at[pl.ds(base, BPW)])
    pltpu.sync_copy(neg_v, neg_hbm.at[pl.ds(wid * (BPW * NS), BPW * NS)])

_sc_logits = pl.kernel(
    _sc_logits_body,
    out_type=[jax.ShapeDtypeStruct((B,), jnp.float32),
              jax.ShapeDtypeStruct((B * NS,), jnp.float32)],
    mesh=plsc.VectorSubcoreMesh(core_axis_name="c", subcore_axis_name="s"),
    compiler_params=pltpu.CompilerParams(needs_layout_passes=False, use_tc_tiling_on_sc=False),
    scratch_types=[
        pltpu.VMEM((BPW,), jnp.int32),
        pltpu.VMEM((BPW * C,), jnp.int32),
        pltpu.VMEM((BPW * NS,), jnp.int32),
        pltpu.VMEM((2, 16, D), jnp.float32),
        pltpu.VMEM((2, 160, D), jnp.float32),
        pltpu.VMEM((2, 320, D), jnp.float32),
        pltpu.VMEM((BPW,), jnp.float32),
        pltpu.VMEM((BPW * NS,), jnp.float32),
        pltpu.SemaphoreType.DMA,
        pltpu.SemaphoreType.DMA,
    ],
)

def _logsig(x):
    return jnp.minimum(x, 0.0) - jnp.log1p(jnp.exp(-jnp.abs(x)))


def _loss_body(pos_ref, neg_ref, pr_ref, o_ref):
    o_ref[0, 0] = (jnp.sum(_logsig(pos_ref[...]))
                   + jnp.sum(_logsig(-neg_ref[...]))
                   + (-0.5) * jnp.sum(pr_ref[...])
                   + jnp.float32(PRIOR_CONST))


def kernel(target_ixs, context_ixs, negative_sample_ixs, W_word, W_ctx):
    (pr,) = _sc_prior(W_word, W_ctx)
    pos, neg = _sc_logits(
        W_word, W_ctx,
        target_ixs.astype(jnp.int32),
        context_ixs.astype(jnp.int32).reshape(-1),
        negative_sample_ixs.astype(jnp.int32).reshape(-1),
    )
    ll = pl.pallas_call(
        _loss_body,
        out_specs=pl.BlockSpec(memory_space=pltpu.SMEM),
        out_shape=jax.ShapeDtypeStruct((1, 1), jnp.float32),
    )(pos.reshape(128, 128), neg.reshape(2560, 128), pr.reshape(4, 128))
    return ll[0, 0]


# speedup vs baseline: 1.0798x; 1.0213x over previous
import math

import jax
import jax.numpy as jnp
from jax import lax
from jax.experimental import pallas as pl
from jax.experimental.pallas import tpu as pltpu
from jax.experimental.pallas import tpu_sc as plsc

V, D, B, C, NS = 100000, 64, 16384, 10, 20
NW = 32
BPW = B // NW
NG = BPW // 16
PRIOR_CONST = 2.0 * V * D * (-0.5 * math.log(2.0 * math.pi))


ROWS_PW = V // NW
PCHUNK = 125
NCHUNK = ROWS_PW // PCHUNK


def _sc_logits_body(w_word, w_ctx, tgt_hbm, ctxf_hbm, negf_hbm,
                    pos_hbm, neg_hbm, prior_hbm,
                    tgt_idx, ctx_idx, neg_idx,
                    t_3d, c_3d, n_3d, pos_v, neg_v, pr_3d, pr_v, sem0, sem1):
    wid = lax.axis_index("s") * 2 + lax.axis_index("c")
    base = wid * BPW
    pltpu.sync_copy(tgt_hbm.at[pl.ds(base, BPW)], tgt_idx)
    pltpu.sync_copy(ctxf_hbm.at[pl.ds(wid * (BPW * C), BPW * C)], ctx_idx)
    pltpu.sync_copy(negf_hbm.at[pl.ds(wid * (BPW * NS), BPW * NS)], neg_idx)

    lane = lax.broadcasted_iota(jnp.int32, (16,), 0)
    zero = jnp.zeros((16,), jnp.float32)

    bufs = ((t_3d.at[0], c_3d.at[0], n_3d.at[0], sem0),
            (t_3d.at[1], c_3d.at[1], n_3d.at[1], sem1))

    def descs(g, slot):
        t_b, c_b, n_b, sem = bufs[slot]
        o16 = pl.multiple_of(g * 16, 16)
        o160 = pl.multiple_of(g * 160, 32)
        o320 = pl.multiple_of(g * 320, 64)
        return (
            pltpu.make_async_copy(w_word.at[tgt_idx.at[pl.ds(o16, 16)]], t_b, sem),
            pltpu.make_async_copy(w_ctx.at[ctx_idx.at[pl.ds(o160, 80)]],
                                  c_b.at[pl.ds(0, 80)], sem),
            pltpu.make_async_copy(w_ctx.at[ctx_idx.at[pl.ds(o160 + 80, 80)]],
                                  c_b.at[pl.ds(80, 80)], sem),
            pltpu.make_async_copy(w_word.at[neg_idx.at[pl.ds(o320, 128)]],
                                  n_b.at[pl.ds(0, 128)], sem),
            pltpu.make_async_copy(w_word.at[neg_idx.at[pl.ds(o320 + 128, 128)]],
                                  n_b.at[pl.ds(128, 128)], sem),
            pltpu.make_async_copy(w_word.at[neg_idx.at[pl.ds(o320 + 256, 64)]],
                                  n_b.at[pl.ds(256, 64)], sem),
        )

    def issue(g, slot):
        for cp in descs(g, slot):
            cp.start()

    def wait(g, slot):
        for cp in descs(g, slot):
            cp.wait()

    NV = D // 16

    def compute(g, slot):
        t_b, c_b, n_b, _ = bufs[slot]
        o16 = pl.multiple_of(g * 16, 16)

        def per_b(bb, carry):
            p_pack, n_pack = carry
            m = lane == jnp.full((16,), bb, jnp.int32)
            crow = bb * C
            nrow = bb * NS
            c_sum = []
            for k4 in range(NV):
                acc = c_b[crow, pl.ds(k4 * 16, 16)]
                for j in range(1, C):
                    acc = acc + c_b[crow + j, pl.ds(k4 * 16, 16)]
                c_sum.append(acc)
            pv = t_b[bb, pl.ds(0, 16)] * c_sum[0]
            for k4 in range(1, NV):
                pv = pv + t_b[bb, pl.ds(k4 * 16, 16)] * c_sum[k4]
            p_pack = jnp.where(m, jnp.sum(pv), p_pack)
            new_n = []
            for k in range(NS):
                nv = n_b[nrow + k, pl.ds(0, 16)] * c_sum[0]
                for k4 in range(1, NV):
                    nv = nv + n_b[nrow + k, pl.ds(k4 * 16, 16)] * c_sum[k4]
                new_n.append(jnp.where(m, jnp.sum(nv), n_pack[k]))
            return p_pack, tuple(new_n)

        p, n = lax.fori_loop(0, 16, per_b, (zero, (zero,) * NS))
        pos_v[pl.ds(o16, 16)] = p
        for k in range(NS):
            neg_v[pl.ds(k * BPW + o16, 16)] = n[k]

    issue(0, 0)
    issue(1, 1)

    def group_pair(it, _):
        g0 = it * 2
        wait(g0, 0)
        compute(g0, 0)

        @pl.when(g0 + 2 < NG)
        def _():
            issue(g0 + 2, 0)

        wait(g0 + 1, 1)
        compute(g0 + 1, 1)

        @pl.when(g0 + 3 < NG)
        def _():
            issue(g0 + 3, 1)

        return ()

    lax.fori_loop(0, NG // 2, group_pair, ())
    pltpu.sync_copy(pos_v, pos_hbm.at[pl.ds(base, BPW)])
    pltpu.sync_copy(neg_v, neg_hbm.at[pl.ds(wid * (BPW * NS), BPW * NS)])

    r0 = wid * ROWS_PW

    def prior_issue(c, slot):
        tbl = w_word if c < NCHUNK else w_ctx
        roff = r0 + (c % NCHUNK) * PCHUNK
        sem = sem0 if slot == 0 else sem1
        return pltpu.make_async_copy(tbl.at[pl.ds(roff, PCHUNK), :],
                                     pr_3d.at[slot], sem)

    def prior_sum(slot, acc):
        pr_b = pr_3d.at[slot]

        def row16(i, a):
            v = pr_b[i, pl.ds(0, 16)]
            a0 = a[0] + v * v
            v = pr_b[i, pl.ds(16, 16)]
            a1 = a[1] + v * v
            v = pr_b[i, pl.ds(32, 16)]
            a2 = a[2] + v * v
            v = pr_b[i, pl.ds(48, 16)]
            a3 = a[3] + v * v
            return (a0, a1, a2, a3)

        return lax.fori_loop(0, PCHUNK, row16, acc)

    for c in range(2 * NCHUNK):
        if c == 0:
            prior_issue(0, 0).start()
        if c + 1 < 2 * NCHUNK:
            prior_issue(c + 1, (c + 1) % 2).start()
        prior_issue(c, c % 2).wait()
        if c == 0:
            acc = ((zero,) * 4)
        acc = prior_sum(c % 2, acc)
    pr_v[pl.ds(0, 16)] = acc[0] + acc[1] + acc[2] + acc[3]
    pltpu.sync_copy(pr_v, prior_hbm.at[pl.ds(wid * 16, 16)])


_sc_logits = pl.kernel(
    _sc_logits_body,
    out_type=[jax.ShapeDtypeStruct((B,), jnp.float32),
              jax.ShapeDtypeStruct((B * NS,), jnp.float32),
              jax.ShapeDtypeStruct((NW * 16,), jnp.float32)],
    mesh=plsc.VectorSubcoreMesh(core_axis_name="c", subcore_axis_name="s"),
    compiler_params=pltpu.CompilerParams(needs_layout_passes=False, use_tc_tiling_on_sc=False),
    scratch_types=[
        pltpu.VMEM((BPW,), jnp.int32),
        pltpu.VMEM((BPW * C,), jnp.int32),
        pltpu.VMEM((BPW * NS,), jnp.int32),
        pltpu.VMEM((2, 16, D), jnp.float32),
        pltpu.VMEM((2, 160, D), jnp.float32),
        pltpu.VMEM((2, 320, D), jnp.float32),
        pltpu.VMEM((BPW,), jnp.float32),
        pltpu.VMEM((BPW * NS,), jnp.float32),
        pltpu.VMEM((2, PCHUNK, D), jnp.float32),
        pltpu.VMEM((16,), jnp.float32),
        pltpu.SemaphoreType.DMA,
        pltpu.SemaphoreType.DMA,
    ],
)

def _logsig(x):
    return jnp.minimum(x, 0.0) - jnp.log1p(jnp.exp(-jnp.abs(x)))


def _loss_body(pos_ref, neg_ref, pr_ref, o_ref):
    o_ref[0, 0] = (jnp.sum(_logsig(pos_ref[...]))
                   + jnp.sum(_logsig(-neg_ref[...]))
                   + (-0.5) * jnp.sum(pr_ref[...])
                   + jnp.float32(PRIOR_CONST))


def kernel(target_ixs, context_ixs, negative_sample_ixs, W_word, W_ctx):
    pos, neg, pr = _sc_logits(
        W_word, W_ctx,
        target_ixs.astype(jnp.int32),
        context_ixs.astype(jnp.int32).reshape(-1),
        negative_sample_ixs.astype(jnp.int32).reshape(-1),
    )
    ll = pl.pallas_call(
        _loss_body,
        out_specs=pl.BlockSpec(memory_space=pltpu.SMEM),
        out_shape=jax.ShapeDtypeStruct((1, 1), jnp.float32),
    )(pos.reshape(128, 128), neg.reshape(2560, 128), pr.reshape(4, 128))
    return ll[0, 0]
